# baseline (speedup 1.0000x reference)
import functools

import jax
import jax.numpy as jnp
from jax.experimental import pallas as pl
from jax.experimental.pallas import tpu as pltpu
from jax.experimental.pallas import tpu_sc as plsc

_D_INP = 3584
_D_HIDE = 65536
_TOP_K = 7
_BATCH = 256
_BLK = 512
_NTOP = 7
_NPAD = 16

_BIG_I32 = 2**30

_SC_LANES = 16
_SC_WORKERS = 32
_SC_CHUNK = 128


def _enc_body(nblk, blk, x_ref, w_ref, be_ref, bd_ref, topv_ref, topi_ref,
              tv, ti, hb):
    j = pl.program_id(0)

    @pl.when(j == 0)
    def _init():
        tv[...] = jnp.full(tv.shape, -jnp.inf, dtype=tv.dtype)
        ti[...] = jnp.zeros(ti.shape, dtype=ti.dtype)

    xc = x_ref[...] - bd_ref[...]
    h = jax.lax.dot_general(
        xc, w_ref[...], (((1,), (0,)), ((), ())),
        preferred_element_type=jnp.float32,
        precision=jax.lax.Precision.DEFAULT,
    ) + be_ref[...]
    hb[...] = h

    b = h.shape[0]
    col = jax.lax.broadcasted_iota(jnp.int32, (b, blk), 1) + j * blk
    lane = jax.lax.broadcasted_iota(jnp.int32, (b, _NTOP), 1)

    def _maxarg():
        hv = hb[...]
        m = jnp.max(hv, axis=1, keepdims=True)
        am = jnp.min(jnp.where(hv == m, col, _BIG_I32), axis=1, keepdims=True)
        return m, am

    m0, am0 = _maxarg()
    go0 = jnp.any(m0 > tv[:, _NTOP - 1:_NTOP])

    def _round(carry):
        m, am, _ = carry
        tvv = tv[...]
        tii = ti[...]
        pos = jnp.sum((tvv >= m).astype(jnp.int32), axis=1, keepdims=True)
        sh_v = jnp.concatenate([tvv[:, :1], tvv[:, :_NTOP - 1]], axis=1)
        sh_i = jnp.concatenate([tii[:, :1], tii[:, :_NTOP - 1]], axis=1)
        nv = jnp.where(lane < pos, tvv, jnp.where(lane == pos, m, sh_v))
        ni = jnp.where(lane < pos, tii, jnp.where(lane == pos, am, sh_i))
        tv[...] = nv
        ti[...] = ni
        hb[...] = jnp.where(col == am, -jnp.inf, hb[...])
        m2, am2 = _maxarg()
        go2 = jnp.any(m2 > nv[:, _NTOP - 1:_NTOP])
        return m2, am2, go2

    jax.lax.while_loop(lambda c: c[2], _round, (m0, am0, go0))

    @pl.when(j == nblk - 1)
    def _emit():
        b_ = tv.shape[0]
        padv = jnp.full((b_, _NPAD - _NTOP), -jnp.inf, dtype=jnp.float32)
        padi = jnp.zeros((b_, _NPAD - _NTOP), dtype=jnp.int32)
        topv_ref[...] = jnp.concatenate([tv[...], padv], axis=1)
        topi_ref[...] = jnp.concatenate([ti[...], padi], axis=1)


def _feat_body(blk, top_k, topv_ref, topi_ref, feat_ref):
    j = pl.program_id(0)
    b = feat_ref.shape[0]
    col = jax.lax.broadcasted_iota(jnp.int32, (b, blk), 1) + j * blk
    f = jnp.zeros((b, blk), dtype=jnp.float32)
    for k in range(top_k):
        v = jax.nn.relu(topv_ref[:, k:k + 1])
        i = topi_ref[:, k:k + 1]
        f = jnp.where(col == i, v, f)
    feat_ref[...] = f


def _splat_lane(vec16, k):
    idx = jnp.full((_SC_LANES, 1), k, dtype=jnp.int32)
    return jax.lax.gather(
        vec16, idx,
        jax.lax.GatherDimensionNumbers(
            offset_dims=(), collapsed_slice_dims=(0,), start_index_map=(0,)),
        (1,), mode=jax.lax.GatherScatterMode.PROMISE_IN_BOUNDS)


def _sc_dec_body(nchunk, d_inp, top_k, rows_per,
                 w1_hbm, tv_hbm, ti_hbm, base_hbm, bd_hbm, o_hbm,
                 tv_v, ti_v, base_v, bd_v, acc_v, idx0, idx1, g0, g1,
                 sem0, sem1):
    wid = jax.lax.axis_index("s") * 2 + jax.lax.axis_index("c")
    rows0 = wid * rows_per
    pltpu.sync_copy(tv_hbm.at[pl.ds(rows0, rows_per)], tv_v)
    pltpu.sync_copy(ti_hbm.at[pl.ds(rows0, rows_per)], ti_v)
    pltpu.sync_copy(base_hbm, base_v)
    pltpu.sync_copy(bd_hbm, bd_v)

    idx_bufs = (idx0, idx1)
    g_bufs = (g0, g1)
    sems = (sem0, sem1)

    def build_and_fire(r, k, kbuf):
        ib = idx_bufs[kbuf]
        col16 = _splat_lane(ti_v[r], k)

        @pl.loop(0, nchunk)
        def _(c):
            for u in range(8):
                s = pl.ds(u * _SC_LANES, _SC_LANES)
                ib[c, s] = base_v[c, s] + col16

        @pl.loop(0, nchunk)
        def _(c):
            pltpu.make_async_copy(
                w1_hbm.at[ib.at[c]],
                g_bufs[kbuf].at[pl.ds(c * _SC_CHUNK, _SC_CHUNK)],
                sems[kbuf]).start()

    def drain_and_fma(r, k, kbuf):
        pltpu.make_async_copy(bd_hbm, g_bufs[kbuf], sems[kbuf]).wait()
        vk16 = jnp.maximum(_splat_lane(tv_v[r], k), 0.0)
        g = g_bufs[kbuf]

        @pl.loop(0, d_inp, step=64)
        def _(c):
            for u in range(4):
                s = pl.ds(c + u * _SC_LANES, _SC_LANES)
                acc_v[s] = acc_v[s] + vk16 * g[s]

    @pl.loop(0, rows_per)
    def _row(r):
        @pl.loop(0, d_inp, step=64)
        def _(c):
            for u in range(4):
                s = pl.ds(c + u * _SC_LANES, _SC_LANES)
                acc_v[s] = bd_v[s]

        build_and_fire(r, 0, 0)
        for k in range(top_k):
            if k + 1 < top_k:
                build_and_fire(r, k + 1, (k + 1) % 2)
            drain_and_fma(r, k, k % 2)
        pltpu.sync_copy(acc_v, o_hbm.at[rows0 + r])


def _sc_decode(topv, topi, W_enc, b_dec):
    batch = topv.shape[0]
    d_inp, d_hide = W_enc.shape
    rows_per = batch // _SC_WORKERS
    nchunk = d_inp // _SC_CHUNK
    w1 = W_enc.reshape(-1)
    base = (jnp.arange(d_inp, dtype=jnp.int32) * d_hide).reshape(
        nchunk, _SC_CHUNK)
    mesh = plsc.VectorSubcoreMesh(core_axis_name="c", subcore_axis_name="s")
    f = pl.kernel(
        functools.partial(_sc_dec_body, nchunk, d_inp, _TOP_K, rows_per),
        out_type=jax.ShapeDtypeStruct((batch, d_inp), jnp.float32),
        mesh=mesh,
        scratch_types=[
            pltpu.VMEM((rows_per, _NPAD), jnp.float32),
            pltpu.VMEM((rows_per, _NPAD), jnp.int32),
            pltpu.VMEM((nchunk, _SC_CHUNK), jnp.int32),
            pltpu.VMEM((d_inp,), jnp.float32),
            pltpu.VMEM((d_inp,), jnp.float32),
            pltpu.VMEM((nchunk, _SC_CHUNK), jnp.int32),
            pltpu.VMEM((nchunk, _SC_CHUNK), jnp.int32),
            pltpu.VMEM((d_inp,), jnp.float32),
            pltpu.VMEM((d_inp,), jnp.float32),
            pltpu.SemaphoreType.DMA,
            pltpu.SemaphoreType.DMA,
        ],
    )
    return f(w1, topv, topi, base, b_dec)


def _run(x, W_enc, b_enc, b_dec, blk, top_k, interpret=False):
    batch, d_inp = x.shape
    d_hide = W_enc.shape[1]
    nblk = d_hide // blk
    be2 = b_enc.reshape(1, d_hide)
    bd2 = b_dec.reshape(1, d_inp)

    topv, topi = pl.pallas_call(
        functools.partial(_enc_body, nblk, blk),
        grid=(nblk,),
        in_specs=[
            pl.BlockSpec((batch, d_inp), lambda j: (0, 0)),
            pl.BlockSpec((d_inp, blk), lambda j: (0, j)),
            pl.BlockSpec((1, blk), lambda j: (0, j)),
            pl.BlockSpec((1, d_inp), lambda j: (0, 0)),
        ],
        out_specs=[
            pl.BlockSpec((batch, _NPAD), lambda j: (0, 0)),
            pl.BlockSpec((batch, _NPAD), lambda j: (0, 0)),
        ],
        out_shape=[
            jax.ShapeDtypeStruct((batch, _NPAD), jnp.float32),
            jax.ShapeDtypeStruct((batch, _NPAD), jnp.int32),
        ],
        scratch_shapes=[
            pltpu.VMEM((batch, _NTOP), jnp.float32),
            pltpu.VMEM((batch, _NTOP), jnp.int32),
            pltpu.VMEM((batch, blk), jnp.float32),
        ],
        interpret=interpret,
    )(x, W_enc, be2, bd2)

    feat = pl.pallas_call(
        functools.partial(_feat_body, blk, top_k),
        grid=(nblk,),
        in_specs=[
            pl.BlockSpec((batch, _NPAD), lambda j: (0, 0)),
            pl.BlockSpec((batch, _NPAD), lambda j: (0, 0)),
        ],
        out_specs=pl.BlockSpec((batch, blk), lambda j: (0, j)),
        out_shape=jax.ShapeDtypeStruct((batch, d_hide), jnp.float32),
        interpret=interpret,
    )(topv, topi)

    recon = _sc_decode(topv, topi, W_enc, b_dec)
    return recon, feat


def kernel(x, W_enc, b_enc, b_dec):
    return _run(x, W_enc, b_enc, b_dec, _BLK, _TOP_K)


# speedup vs baseline: 1.7102x; 1.7102x over previous
import functools

import jax
import jax.numpy as jnp
from jax.experimental import pallas as pl
from jax.experimental.pallas import tpu as pltpu

_D_INP = 3584
_D_HIDE = 65536
_TOP_K = 7
_BATCH = 256
_BLK = 512
_BLK_DEC = 2048
_NTOP = 7

_BIG_I32 = 2**30


def _enc_body(nblk, blk, x_ref, w_ref, be_ref, bd_ref, topv_ref, topi_ref,
              wb_ref, tv, ti, hb):
    j = pl.program_id(0)

    @pl.when(j == 0)
    def _init():
        tv[...] = jnp.full(tv.shape, -jnp.inf, dtype=tv.dtype)
        ti[...] = jnp.zeros(ti.shape, dtype=ti.dtype)

    w = w_ref[...]
    wb_ref[...] = w.astype(jnp.bfloat16)

    xc = x_ref[...] - bd_ref[...]
    h = jax.lax.dot_general(
        xc, w, (((1,), (0,)), ((), ())),
        preferred_element_type=jnp.float32,
        precision=jax.lax.Precision.DEFAULT,
    ) + be_ref[...]
    hb[...] = h

    b = h.shape[0]
    col = jax.lax.broadcasted_iota(jnp.int32, (b, blk), 1) + j * blk
    lane = jax.lax.broadcasted_iota(jnp.int32, (b, _NTOP), 1)

    m0 = jnp.max(h, axis=1, keepdims=True)
    am0 = jnp.min(jnp.where(h == m0, col, _BIG_I32), axis=1, keepdims=True)
    go0 = jnp.any(m0 > tv[:, _NTOP - 1:_NTOP])

    def _round(carry):
        m, am, _ = carry
        tvv = tv[...]
        tii = ti[...]
        pos = jnp.sum((tvv >= m).astype(jnp.int32), axis=1, keepdims=True)
        sh_v = jnp.concatenate([tvv[:, :1], tvv[:, :_NTOP - 1]], axis=1)
        sh_i = jnp.concatenate([tii[:, :1], tii[:, :_NTOP - 1]], axis=1)
        nv = jnp.where(lane < pos, tvv, jnp.where(lane == pos, m, sh_v))
        ni = jnp.where(lane < pos, tii, jnp.where(lane == pos, am, sh_i))
        tv[...] = nv
        ti[...] = ni
        killed = jnp.where(col == am, -jnp.inf, hb[...])
        hb[...] = killed
        m2 = jnp.max(killed, axis=1, keepdims=True)
        am2 = jnp.min(jnp.where(killed == m2, col, _BIG_I32), axis=1,
                      keepdims=True)
        go2 = jnp.any(m2 > nv[:, _NTOP - 1:_NTOP])
        return m2, am2, go2

    jax.lax.while_loop(lambda c: c[2], _round, (m0, am0, go0))

    @pl.when(j == nblk - 1)
    def _emit():
        topv_ref[...] = tv[...]
        topi_ref[...] = ti[...]


def _dec_body(nblk, blk, top_k, topv_ref, topi_ref, wb_ref, bd_ref,
              feat_ref, recon_ref, acc):
    j = pl.program_id(0)
    b = feat_ref.shape[0]
    col = jax.lax.broadcasted_iota(jnp.int32, (b, blk), 1) + j * blk

    f = jnp.zeros((b, blk), dtype=jnp.float32)
    for k in range(top_k):
        v = jax.nn.relu(topv_ref[:, k:k + 1])
        i = topi_ref[:, k:k + 1]
        f = jnp.where(col == i, v, f)
    feat_ref[...] = f

    contrib = jax.lax.dot_general(
        f.astype(jnp.bfloat16), wb_ref[...], (((1,), (1,)), ((), ())),
        preferred_element_type=jnp.float32,
        precision=jax.lax.Precision.DEFAULT,
    )

    @pl.when(j == 0)
    def _init():
        acc[...] = jnp.zeros(acc.shape, dtype=acc.dtype)

    acc[...] += contrib

    @pl.when(j == nblk - 1)
    def _emit():
        recon_ref[...] = acc[...] + bd_ref[...]


def _run(x, W_enc, b_enc, b_dec, blk, blk_dec, top_k, interpret=False):
    batch, d_inp = x.shape
    d_hide = W_enc.shape[1]
    nblk = d_hide // blk
    nblk_dec = d_hide // blk_dec
    be2 = b_enc.reshape(1, d_hide)
    bd2 = b_dec.reshape(1, d_inp)

    topv, topi, w_bf16 = pl.pallas_call(
        functools.partial(_enc_body, nblk, blk),
        grid=(nblk,),
        in_specs=[
            pl.BlockSpec((batch, d_inp), lambda j: (0, 0)),
            pl.BlockSpec((d_inp, blk), lambda j: (0, j)),
            pl.BlockSpec((1, blk), lambda j: (0, j)),
            pl.BlockSpec((1, d_inp), lambda j: (0, 0)),
        ],
        out_specs=[
            pl.BlockSpec((batch, _NTOP), lambda j: (0, 0)),
            pl.BlockSpec((batch, _NTOP), lambda j: (0, 0)),
            pl.BlockSpec((d_inp, blk), lambda j: (0, j)),
        ],
        out_shape=[
            jax.ShapeDtypeStruct((batch, _NTOP), jnp.float32),
            jax.ShapeDtypeStruct((batch, _NTOP), jnp.int32),
            jax.ShapeDtypeStruct((d_inp, d_hide), jnp.bfloat16),
        ],
        scratch_shapes=[
            pltpu.VMEM((batch, _NTOP), jnp.float32),
            pltpu.VMEM((batch, _NTOP), jnp.int32),
            pltpu.VMEM((batch, blk), jnp.float32),
        ],
        interpret=interpret,
    )(x, W_enc, be2, bd2)

    feat, recon = pl.pallas_call(
        functools.partial(_dec_body, nblk_dec, blk_dec, top_k),
        grid=(nblk_dec,),
        in_specs=[
            pl.BlockSpec((batch, _NTOP), lambda j: (0, 0)),
            pl.BlockSpec((batch, _NTOP), lambda j: (0, 0)),
            pl.BlockSpec((d_inp, blk_dec), lambda j: (0, j)),
            pl.BlockSpec((1, d_inp), lambda j: (0, 0)),
        ],
        out_specs=[
            pl.BlockSpec((batch, blk_dec), lambda j: (0, j)),
            pl.BlockSpec((batch, d_inp), lambda j: (0, 0)),
        ],
        out_shape=[
            jax.ShapeDtypeStruct((batch, d_hide), jnp.float32),
            jax.ShapeDtypeStruct((batch, d_inp), jnp.float32),
        ],
        scratch_shapes=[
            pltpu.VMEM((batch, d_inp), jnp.float32),
        ],
        interpret=interpret,
    )(topv, topi, w_bf16, bd2)

    return recon, feat


def kernel(x, W_enc, b_enc, b_dec):
    return _run(x, W_enc, b_enc, b_dec, _BLK, _BLK_DEC, _TOP_K)


# speedup vs baseline: 1.7137x; 1.0021x over previous
import functools

import jax
import jax.numpy as jnp
from jax.experimental import pallas as pl
from jax.experimental.pallas import tpu as pltpu

_D_INP = 3584
_D_HIDE = 65536
_TOP_K = 7
_BATCH = 256
_BLK = 512
_BLK_DEC = 2048
_NTOP = 7

_BIG_I32 = 2**30


def _enc_body(nblk, blk, x_ref, w_ref, be_ref, bd_ref, topv_ref, topi_ref,
              wb_ref, tv, ti, hb, xcb):
    j = pl.program_id(0)

    @pl.when(j == 0)
    def _init():
        tv[...] = jnp.full(tv.shape, -jnp.inf, dtype=tv.dtype)
        ti[...] = jnp.zeros(ti.shape, dtype=ti.dtype)
        xcb[...] = (x_ref[...] - bd_ref[...]).astype(jnp.bfloat16)

    wbv = w_ref[...].astype(jnp.bfloat16)
    wb_ref[...] = wbv

    h = jax.lax.dot_general(
        xcb[...], wbv, (((1,), (0,)), ((), ())),
        preferred_element_type=jnp.float32,
        precision=jax.lax.Precision.DEFAULT,
    ) + be_ref[...]
    hb[...] = h

    b = h.shape[0]
    col = jax.lax.broadcasted_iota(jnp.int32, (b, blk), 1) + j * blk
    lane = jax.lax.broadcasted_iota(jnp.int32, (b, _NTOP), 1)

    m0 = jnp.max(h, axis=1, keepdims=True)
    am0 = jnp.min(jnp.where(h == m0, col, _BIG_I32), axis=1, keepdims=True)
    go0 = jnp.any(m0 > tv[:, _NTOP - 1:_NTOP])

    def _round(carry):
        m, am, _ = carry
        tvv = tv[...]
        tii = ti[...]
        pos = jnp.sum((tvv >= m).astype(jnp.int32), axis=1, keepdims=True)
        sh_v = jnp.concatenate([tvv[:, :1], tvv[:, :_NTOP - 1]], axis=1)
        sh_i = jnp.concatenate([tii[:, :1], tii[:, :_NTOP - 1]], axis=1)
        nv = jnp.where(lane < pos, tvv, jnp.where(lane == pos, m, sh_v))
        ni = jnp.where(lane < pos, tii, jnp.where(lane == pos, am, sh_i))
        tv[...] = nv
        ti[...] = ni
        killed = jnp.where(col == am, -jnp.inf, hb[...])
        hb[...] = killed
        m2 = jnp.max(killed, axis=1, keepdims=True)
        am2 = jnp.min(jnp.where(killed == m2, col, _BIG_I32), axis=1,
                      keepdims=True)
        go2 = jnp.any(m2 > nv[:, _NTOP - 1:_NTOP])
        return m2, am2, go2

    jax.lax.while_loop(lambda c: c[2], _round, (m0, am0, go0))

    @pl.when(j == nblk - 1)
    def _emit():
        topv_ref[...] = tv[...]
        topi_ref[...] = ti[...]


def _dec_body(nblk, blk, top_k, topv_ref, topi_ref, wb_ref, bd_ref,
              feat_ref, recon_ref, acc):
    j = pl.program_id(0)
    b = feat_ref.shape[0]
    col = jax.lax.broadcasted_iota(jnp.int32, (b, blk), 1) + j * blk

    f = jnp.zeros((b, blk), dtype=jnp.float32)
    for k in range(top_k):
        v = jax.nn.relu(topv_ref[:, k:k + 1])
        i = topi_ref[:, k:k + 1]
        f = jnp.where(col == i, v, f)
    feat_ref[...] = f

    contrib = jax.lax.dot_general(
        f.astype(jnp.bfloat16), wb_ref[...], (((1,), (1,)), ((), ())),
        preferred_element_type=jnp.float32,
        precision=jax.lax.Precision.DEFAULT,
    )

    @pl.when(j == 0)
    def _init():
        acc[...] = jnp.zeros(acc.shape, dtype=acc.dtype)

    acc[...] += contrib

    @pl.when(j == nblk - 1)
    def _emit():
        recon_ref[...] = acc[...] + bd_ref[...]


def _run(x, W_enc, b_enc, b_dec, blk, blk_dec, top_k, interpret=False):
    batch, d_inp = x.shape
    d_hide = W_enc.shape[1]
    nblk = d_hide // blk
    nblk_dec = d_hide // blk_dec
    be2 = b_enc.reshape(1, d_hide)
    bd2 = b_dec.reshape(1, d_inp)

    topv, topi, w_bf16 = pl.pallas_call(
        functools.partial(_enc_body, nblk, blk),
        grid=(nblk,),
        in_specs=[
            pl.BlockSpec((batch, d_inp), lambda j: (0, 0)),
            pl.BlockSpec((d_inp, blk), lambda j: (0, j)),
            pl.BlockSpec((1, blk), lambda j: (0, j)),
            pl.BlockSpec((1, d_inp), lambda j: (0, 0)),
        ],
        out_specs=[
            pl.BlockSpec((batch, _NTOP), lambda j: (0, 0)),
            pl.BlockSpec((batch, _NTOP), lambda j: (0, 0)),
            pl.BlockSpec((d_inp, blk), lambda j: (0, j)),
        ],
        out_shape=[
            jax.ShapeDtypeStruct((batch, _NTOP), jnp.float32),
            jax.ShapeDtypeStruct((batch, _NTOP), jnp.int32),
            jax.ShapeDtypeStruct((d_inp, d_hide), jnp.bfloat16),
        ],
        scratch_shapes=[
            pltpu.VMEM((batch, _NTOP), jnp.float32),
            pltpu.VMEM((batch, _NTOP), jnp.int32),
            pltpu.VMEM((batch, blk), jnp.float32),
            pltpu.VMEM((batch, d_inp), jnp.bfloat16),
        ],
        interpret=interpret,
    )(x, W_enc, be2, bd2)

    feat, recon = pl.pallas_call(
        functools.partial(_dec_body, nblk_dec, blk_dec, top_k),
        grid=(nblk_dec,),
        in_specs=[
            pl.BlockSpec((batch, _NTOP), lambda j: (0, 0)),
            pl.BlockSpec((batch, _NTOP), lambda j: (0, 0)),
            pl.BlockSpec((d_inp, blk_dec), lambda j: (0, j)),
            pl.BlockSpec((1, d_inp), lambda j: (0, 0)),
        ],
        out_specs=[
            pl.BlockSpec((batch, blk_dec), lambda j: (0, j)),
            pl.BlockSpec((batch, d_inp), lambda j: (0, 0)),
        ],
        out_shape=[
            jax.ShapeDtypeStruct((batch, d_hide), jnp.float32),
            jax.ShapeDtypeStruct((batch, d_inp), jnp.float32),
        ],
        scratch_shapes=[
            pltpu.VMEM((batch, d_inp), jnp.float32),
        ],
        interpret=interpret,
    )(topv, topi, w_bf16, bd2)

    return recon, feat


def kernel(x, W_enc, b_enc, b_dec):
    return _run(x, W_enc, b_enc, b_dec, _BLK, _BLK_DEC, _TOP_K)
